# baseline (speedup 1.0000x reference)
import functools

import jax
import jax.numpy as jnp
from jax import lax
from jax.experimental import pallas as pl
from jax.experimental.pallas import tpu as pltpu
from jax.experimental.pallas import tpu_sc as plsc

B = 4096
E = 131072
NPLANE = 5

NC = 2
NS = 16
NW = NC * NS

ROWS_PER_TILE = B // NW
CELLS = ROWS_PER_TILE * B
WPP = CELLS // 32
BMW = NPLANE * WPP

ECH = 4096
NCH = E // ECH
LANES = 16


def _any(m):
    return plsc.all_reduce_population_count(m)[0] > 0


def _rmw_or(bm, w, bitv, m):
    def round_(lost):
        old = plsc.load_gather(bm, [w], mask=lost)
        plsc.store_scatter(bm, [w], old | bitv, mask=lost)
        chk = plsc.load_gather(bm, [w], mask=lost)
        return lost & ((chk & bitv) != bitv)

    lost = round_(m)
    lax.while_loop(_any, round_, lost)


def _scan_body(e0, e1, e2, e3, e4, out, ebuf_a, ebuf_b, accbuf, bm, sem_a, sem_b):
    wid = lax.axis_index("s") * NC + lax.axis_index("c")

    def zero(i, _):
        bm[pl.ds(i * LANES, LANES)] = jnp.zeros((LANES,), jnp.int32)
        return 0

    lax.fori_loop(0, BMW // LANES, zero, 0)

    one = jnp.full((LANES,), 1, jnp.int32)

    def process(ebuf, pbase):
        def it(t, lostacc):
            r = ebuf[0, pl.ds(t * LANES, LANES)]
            c = ebuf[1, pl.ds(t * LANES, LANES)]
            for x, y in ((r, c), (c, r)):
                m = (x >> 7) == wid
                loc = ((x & 127) << 12) | y
                w = pbase + (loc >> 5)
                bitv = jnp.left_shift(one, loc & 31)
                old = plsc.load_gather(bm, [w], mask=m)
                plsc.store_scatter(bm, [w], old | bitv, mask=m)
                chk = plsc.load_gather(bm, [w], mask=m)
                lostacc = lostacc | (m & ((chk & bitv) != bitv))
            return lostacc

        lost = lax.fori_loop(
            0, ECH // LANES, it, jnp.zeros((LANES,), jnp.bool_), unroll=2
        )

        @pl.when(_any(lost))
        def _():
            def it2(t, _):
                r = ebuf[0, pl.ds(t * LANES, LANES)]
                c = ebuf[1, pl.ds(t * LANES, LANES)]
                for x, y in ((r, c), (c, r)):
                    m = (x >> 7) == wid
                    loc = ((x & 127) << 12) | y
                    w = pbase + (loc >> 5)
                    bitv = jnp.left_shift(one, loc & 31)
                    _rmw_or(bm, w, bitv, m)
                return 0

            lax.fori_loop(0, ECH // LANES, it2, 0)

    for li, e in enumerate((e0, e1, e2, e3, e4)):
        pbase = li * WPP
        pltpu.async_copy(e.at[:, pl.ds(0, ECH)], ebuf_a, sem_a)

        def pair(pi, _):
            i0 = pi * 2
            pltpu.make_async_copy(
                e.at[:, pl.ds(i0 * ECH, ECH)], ebuf_a, sem_a
            ).wait()
            pltpu.async_copy(e.at[:, pl.ds((i0 + 1) * ECH, ECH)], ebuf_b, sem_b)
            process(ebuf_a, pbase)
            pltpu.make_async_copy(
                e.at[:, pl.ds((i0 + 1) * ECH, ECH)], ebuf_b, sem_b
            ).wait()

            @pl.when(i0 + 2 < NCH)
            def _():
                pltpu.async_copy(
                    e.at[:, pl.ds((i0 + 2) * ECH, ECH)], ebuf_a, sem_a
                )

            process(ebuf_b, pbase)
            return 0

        lax.fori_loop(0, NCH // 2, pair, 0)

    def halfpop(v):
        v = v - ((v >> 1) & 0x55555555)
        v = (v & 0x33333333) + ((v >> 2) & 0x33333333)
        return (v + (v >> 4)) & 0x0F0F0F0F

    def red(g, acc):
        o = g * LANES
        a0 = bm[pl.ds(o, LANES)]
        a1 = bm[pl.ds(WPP + o, LANES)]
        a2 = bm[pl.ds(2 * WPP + o, LANES)]
        s0 = bm[pl.ds(3 * WPP + o, LANES)]
        s1 = bm[pl.ds(4 * WPP + o, LANES)]
        d0 = a0 & ~a1 & ~a2
        d1 = a1 & ~a0 & ~a2
        tot = halfpop(d0 ^ s0) + halfpop(d1 ^ s1)
        return acc + ((tot * 0x01010101) >> 24)

    acc = lax.fori_loop(0, WPP // LANES, red, jnp.zeros((LANES,), jnp.int32))
    accbuf[...] = acc
    pltpu.sync_copy(accbuf, out.at[wid])


@functools.cache
def _make_scan():
    return pl.kernel(
        _scan_body,
        out_type=jax.ShapeDtypeStruct((NW, LANES), jnp.int32),
        mesh=plsc.VectorSubcoreMesh(
            core_axis_name="c",
            subcore_axis_name="s",
            num_cores=NC,
            num_subcores=NS,
        ),
        compiler_params=pltpu.CompilerParams(needs_layout_passes=False),
        scratch_types=[
            pltpu.VMEM((2, ECH), jnp.int32),
            pltpu.VMEM((2, ECH), jnp.int32),
            pltpu.VMEM((LANES,), jnp.int32),
            pltpu.VMEM((BMW,), jnp.int32),
            pltpu.SemaphoreType.DMA,
            pltpu.SemaphoreType.DMA,
        ],
    )


def _finish_body(x_ref, o_ref):
    o_ref[0, 0] = jnp.sum(x_ref[...]).astype(jnp.float32) * (1.0 / (2 * B * B))


_finish = pl.pallas_call(
    _finish_body,
    out_specs=pl.BlockSpec(memory_space=pltpu.SMEM),
    out_shape=jax.ShapeDtypeStruct((1, 1), jnp.float32),
)


def kernel(alpha, beta, edge_index0, edge_index1, edge_index2, score0, score1):
    del alpha, beta
    edges = [
        e.astype(jnp.int32)
        for e in (edge_index0, edge_index1, edge_index2, score0, score1)
    ]
    counts = _make_scan()(*edges)
    return _finish(counts)[0, 0]


# speedup vs baseline: 1.6377x; 1.6377x over previous
import functools

import jax
import jax.numpy as jnp
from jax import lax
from jax.experimental import pallas as pl
from jax.experimental.pallas import tpu as pltpu
from jax.experimental.pallas import tpu_sc as plsc

B = 4096
E = 131072

NC = 2
NS = 16
NW = NC * NS

NSTRIP = 16
ROWS = B // NSTRIP
WPP = ROWS * B // 32
BMW = 3 * WPP

ECH = 4096
NCH = E // ECH
LANES = 16
RCH = 2048


def _any(m):
    return plsc.all_reduce_population_count(m)[0] > 0


def _rmw_or(bm, w, bitv, m):
    def round_(lost):
        old = plsc.load_gather(bm, [w], mask=lost)
        plsc.store_scatter(bm, [w], old | bitv, mask=lost)
        chk = plsc.load_gather(bm, [w], mask=lost)
        return lost & ((chk & bitv) != bitv)

    lost = round_(m)
    lax.while_loop(_any, round_, lost)


def _halfpop(v):
    v = v - ((v >> 1) & 0x55555555)
    v = (v & 0x33333333) + ((v >> 2) & 0x33333333)
    return (v + (v >> 4)) & 0x0F0F0F0F


def _scan_body(
    e0, e1, e2, e3, e4, out, ebuf_a, ebuf_b, accbuf, bm, st0, st1, st2, spb, sem_a, sem_b
):
    cid = lax.axis_index("c")
    sub = lax.axis_index("s")
    wid = sub * NC + cid
    pair = sub >> 1
    strip = cid * 8 + pair
    role = sub & 1

    def zero(i, _):
        bm[pl.ds(i * LANES, LANES)] = jnp.zeros((LANES,), jnp.int32)
        return 0

    lax.fori_loop(0, BMW // LANES, zero, 0)

    one = jnp.full((LANES,), 1, jnp.int32)

    def process(ebuf, pbase):
        def it(t, lostacc):
            r = ebuf[0, pl.ds(t * LANES, LANES)]
            c = ebuf[1, pl.ds(t * LANES, LANES)]
            for x, y in ((r, c), (c, r)):
                m = (x >> 8) == strip
                loc = ((x & 255) << 12) | y
                w = pbase + (loc >> 5)
                bitv = jnp.left_shift(one, loc & 31)
                old = plsc.load_gather(bm, [w], mask=m)
                plsc.store_scatter(bm, [w], old | bitv, mask=m)
                chk = plsc.load_gather(bm, [w], mask=m)
                lostacc = lostacc | (m & ((chk & bitv) != bitv))
            return lostacc

        lost = lax.fori_loop(
            0, ECH // LANES, it, jnp.zeros((LANES,), jnp.bool_), unroll=2
        )

        @pl.when(_any(lost))
        def _():
            def it2(t, _):
                r = ebuf[0, pl.ds(t * LANES, LANES)]
                c = ebuf[1, pl.ds(t * LANES, LANES)]
                for x, y in ((r, c), (c, r)):
                    m = (x >> 8) == strip
                    loc = ((x & 255) << 12) | y
                    w = pbase + (loc >> 5)
                    bitv = jnp.left_shift(one, loc & 31)
                    _rmw_or(bm, w, bitv, m)
                return 0

            lax.fori_loop(0, ECH // LANES, it2, 0)

    def scan_list(e, pbase, lo, hi):
        pltpu.async_copy(e.at[:, pl.ds(lo * ECH, ECH)], ebuf_a, sem_a)

        def pairstep(pi, _):
            i0 = lo + pi * 2
            pltpu.make_async_copy(
                e.at[:, pl.ds(i0 * ECH, ECH)], ebuf_a, sem_a
            ).wait()
            pltpu.async_copy(e.at[:, pl.ds((i0 + 1) * ECH, ECH)], ebuf_b, sem_b)
            process(ebuf_a, pbase)
            pltpu.make_async_copy(
                e.at[:, pl.ds((i0 + 1) * ECH, ECH)], ebuf_b, sem_b
            ).wait()

            @pl.when(i0 + 2 < hi)
            def _():
                pltpu.async_copy(
                    e.at[:, pl.ds((i0 + 2) * ECH, ECH)], ebuf_a, sem_a
                )

            process(ebuf_b, pbase)
            return 0

        lax.fori_loop(0, (hi - lo) // 2, pairstep, 0)

    @pl.when(role == 0)
    def _():
        scan_list(e0, 0, 0, NCH)
        scan_list(e1, WPP, 0, NCH)
        scan_list(e2, 2 * WPP, 0, NCH // 2)

    @pl.when(role == 1)
    def _():
        scan_list(e3, 0, 0, NCH)
        scan_list(e4, WPP, 0, NCH)
        scan_list(e2, 2 * WPP, NCH // 2, NCH)

    @pl.when(role == 1)
    def _():
        pltpu.sync_copy(bm, spb.at[strip])
        accbuf[...] = jnp.zeros((LANES,), jnp.int32)
        pltpu.sync_copy(accbuf, out.at[wid])

    plsc.subcore_barrier()

    @pl.when(role == 0)
    def _():
        def chunk(ch, acc):
            off = ch * RCH
            pltpu.sync_copy(spb.at[strip, pl.ds(off, RCH)], st0)
            pltpu.sync_copy(spb.at[strip, pl.ds(WPP + off, RCH)], st1)
            pltpu.sync_copy(spb.at[strip, pl.ds(2 * WPP + off, RCH)], st2)

            def red(g, acc2):
                o = g * LANES
                a0 = bm[pl.ds(off + o, LANES)]
                a1 = bm[pl.ds(WPP + off + o, LANES)]
                a2 = bm[pl.ds(2 * WPP + off + o, LANES)] | st2[pl.ds(o, LANES)]
                s0 = st0[pl.ds(o, LANES)]
                s1 = st1[pl.ds(o, LANES)]
                d0 = a0 & ~a1 & ~a2
                d1 = a1 & ~a0 & ~a2
                tot = _halfpop(d0 ^ s0) + _halfpop(d1 ^ s1)
                return acc2 + ((tot * 0x01010101) >> 24)

            return lax.fori_loop(0, RCH // LANES, red, acc)

        acc = lax.fori_loop(0, WPP // RCH, chunk, jnp.zeros((LANES,), jnp.int32))
        accbuf[...] = acc
        pltpu.sync_copy(accbuf, out.at[wid])


@functools.cache
def _make_scan():
    return pl.kernel(
        _scan_body,
        out_type=jax.ShapeDtypeStruct((NW, LANES), jnp.int32),
        mesh=plsc.VectorSubcoreMesh(
            core_axis_name="c",
            subcore_axis_name="s",
            num_cores=NC,
            num_subcores=NS,
        ),
        compiler_params=pltpu.CompilerParams(needs_layout_passes=False),
        scratch_types=[
            pltpu.VMEM((2, ECH), jnp.int32),
            pltpu.VMEM((2, ECH), jnp.int32),
            pltpu.VMEM((LANES,), jnp.int32),
            pltpu.VMEM((BMW,), jnp.int32),
            pltpu.VMEM((RCH,), jnp.int32),
            pltpu.VMEM((RCH,), jnp.int32),
            pltpu.VMEM((RCH,), jnp.int32),
            pltpu.HBM((NSTRIP, BMW), jnp.int32),
            pltpu.SemaphoreType.DMA,
            pltpu.SemaphoreType.DMA,
        ],
    )


def _finish_body(x_ref, o_ref):
    o_ref[0, 0] = jnp.sum(x_ref[...]).astype(jnp.float32) * (1.0 / (2 * B * B))


_finish = pl.pallas_call(
    _finish_body,
    out_specs=pl.BlockSpec(memory_space=pltpu.SMEM),
    out_shape=jax.ShapeDtypeStruct((1, 1), jnp.float32),
)


def kernel(alpha, beta, edge_index0, edge_index1, edge_index2, score0, score1):
    del alpha, beta
    edges = [
        e.astype(jnp.int32)
        for e in (edge_index0, edge_index1, edge_index2, score0, score1)
    ]
    counts = _make_scan()(*edges)
    return _finish(counts)[0, 0]


# speedup vs baseline: 1.6798x; 1.0257x over previous
import functools

import jax
import jax.numpy as jnp
from jax import lax
from jax.experimental import pallas as pl
from jax.experimental.pallas import tpu as pltpu
from jax.experimental.pallas import tpu_sc as plsc

B = 4096
E = 131072

NC = 2
NS = 16
NW = NC * NS

NSTRIP = 16
ROWS = B // NSTRIP
WPP = ROWS * B // 32
BMW = 3 * WPP

ECH = 4096
NCH = E // ECH
LANES = 16
RCH = 2048


def _any(m):
    return plsc.all_reduce_population_count(m)[0] > 0


def _rmw_or(bm, w, bitv, m):
    def round_(lost):
        old = plsc.load_gather(bm, [w], mask=lost)
        plsc.store_scatter(bm, [w], old | bitv, mask=lost)
        chk = plsc.load_gather(bm, [w], mask=lost)
        return lost & ((chk & bitv) != bitv)

    lost = round_(m)
    lax.while_loop(_any, round_, lost)


def _halfpop(v):
    v = v - ((v >> 1) & 0x55555555)
    v = (v & 0x33333333) + ((v >> 2) & 0x33333333)
    return (v + (v >> 4)) & 0x0F0F0F0F


def _scan_body(
    e0, e1, e2, e3, e4, out, ebuf_a, ebuf_b, accbuf, bm, st0, st1, st2, spb, sem_a, sem_b
):
    cid = lax.axis_index("c")
    sub = lax.axis_index("s")
    wid = sub * NC + cid
    pair = sub >> 1
    strip = cid * 8 + pair
    role = sub & 1

    def zero(i, _):
        bm[pl.ds(i * LANES, LANES)] = jnp.zeros((LANES,), jnp.int32)
        return 0

    lax.fori_loop(0, BMW // LANES, zero, 0)

    one = jnp.full((LANES,), 1, jnp.int32)

    def process(ebuf, pbase):
        def it(t, lostacc):
            r = ebuf[0, pl.ds(t * LANES, LANES)]
            c = ebuf[1, pl.ds(t * LANES, LANES)]
            for x, y in ((r, c), (c, r)):
                m = (x >> 8) == strip
                w = pbase | ((x & 255) << 7) | (y >> 5)
                bitv = jnp.left_shift(one, y & 31)
                old = plsc.load_gather(bm, [w], mask=m)
                plsc.store_scatter(bm, [w], old | bitv, mask=m)
                chk = plsc.load_gather(bm, [w], mask=m)
                lostacc = lostacc | (m & ((chk & bitv) != bitv))
            return lostacc

        lost = lax.fori_loop(
            0, ECH // LANES, it, jnp.zeros((LANES,), jnp.bool_), unroll=2
        )

        @pl.when(_any(lost))
        def _():
            def it2(t, _):
                r = ebuf[0, pl.ds(t * LANES, LANES)]
                c = ebuf[1, pl.ds(t * LANES, LANES)]
                for x, y in ((r, c), (c, r)):
                    m = (x >> 8) == strip
                    w = pbase | ((x & 255) << 7) | (y >> 5)
                    bitv = jnp.left_shift(one, y & 31)
                    _rmw_or(bm, w, bitv, m)
                return 0

            lax.fori_loop(0, ECH // LANES, it2, 0)

    def scan_list(e, pbase, lo, hi):
        pltpu.async_copy(e.at[:, pl.ds(lo * ECH, ECH)], ebuf_a, sem_a)

        def pairstep(pi, _):
            i0 = lo + pi * 2
            pltpu.make_async_copy(
                e.at[:, pl.ds(i0 * ECH, ECH)], ebuf_a, sem_a
            ).wait()
            pltpu.async_copy(e.at[:, pl.ds((i0 + 1) * ECH, ECH)], ebuf_b, sem_b)
            process(ebuf_a, pbase)
            pltpu.make_async_copy(
                e.at[:, pl.ds((i0 + 1) * ECH, ECH)], ebuf_b, sem_b
            ).wait()

            @pl.when(i0 + 2 < hi)
            def _():
                pltpu.async_copy(
                    e.at[:, pl.ds((i0 + 2) * ECH, ECH)], ebuf_a, sem_a
                )

            process(ebuf_b, pbase)
            return 0

        lax.fori_loop(0, (hi - lo) // 2, pairstep, 0)

    @pl.when(role == 0)
    def _():
        scan_list(e0, 0, 0, NCH)
        scan_list(e1, WPP, 0, NCH)
        scan_list(e2, 2 * WPP, 0, NCH // 2)

    @pl.when(role == 1)
    def _():
        scan_list(e3, 0, 0, NCH)
        scan_list(e4, WPP, 0, NCH)
        scan_list(e2, 2 * WPP, NCH // 2, NCH)

    @pl.when(role == 1)
    def _():
        pltpu.sync_copy(bm, spb.at[strip])
        accbuf[...] = jnp.zeros((LANES,), jnp.int32)
        pltpu.sync_copy(accbuf, out.at[wid])

    plsc.subcore_barrier()

    @pl.when(role == 0)
    def _():
        def chunk(ch, acc):
            off = ch * RCH
            pltpu.sync_copy(spb.at[strip, pl.ds(off, RCH)], st0)
            pltpu.sync_copy(spb.at[strip, pl.ds(WPP + off, RCH)], st1)
            pltpu.sync_copy(spb.at[strip, pl.ds(2 * WPP + off, RCH)], st2)

            def red(g, acc2):
                o = g * LANES
                a0 = bm[pl.ds(off + o, LANES)]
                a1 = bm[pl.ds(WPP + off + o, LANES)]
                a2 = bm[pl.ds(2 * WPP + off + o, LANES)] | st2[pl.ds(o, LANES)]
                s0 = st0[pl.ds(o, LANES)]
                s1 = st1[pl.ds(o, LANES)]
                d0 = a0 & ~a1 & ~a2
                d1 = a1 & ~a0 & ~a2
                tot = _halfpop(d0 ^ s0) + _halfpop(d1 ^ s1)
                return acc2 + ((tot * 0x01010101) >> 24)

            return lax.fori_loop(0, RCH // LANES, red, acc)

        acc = lax.fori_loop(0, WPP // RCH, chunk, jnp.zeros((LANES,), jnp.int32))
        accbuf[...] = acc
        pltpu.sync_copy(accbuf, out.at[wid])


@functools.cache
def _make_scan():
    return pl.kernel(
        _scan_body,
        out_type=jax.ShapeDtypeStruct((NW, LANES), jnp.int32),
        mesh=plsc.VectorSubcoreMesh(
            core_axis_name="c",
            subcore_axis_name="s",
            num_cores=NC,
            num_subcores=NS,
        ),
        compiler_params=pltpu.CompilerParams(needs_layout_passes=False),
        scratch_types=[
            pltpu.VMEM((2, ECH), jnp.int32),
            pltpu.VMEM((2, ECH), jnp.int32),
            pltpu.VMEM((LANES,), jnp.int32),
            pltpu.VMEM((BMW,), jnp.int32),
            pltpu.VMEM((RCH,), jnp.int32),
            pltpu.VMEM((RCH,), jnp.int32),
            pltpu.VMEM((RCH,), jnp.int32),
            pltpu.HBM((NSTRIP, BMW), jnp.int32),
            pltpu.SemaphoreType.DMA,
            pltpu.SemaphoreType.DMA,
        ],
    )


def _finish_body(x_ref, o_ref):
    o_ref[0, 0] = jnp.sum(x_ref[...]).astype(jnp.float32) * (1.0 / (2 * B * B))


_finish = pl.pallas_call(
    _finish_body,
    out_specs=pl.BlockSpec(memory_space=pltpu.SMEM),
    out_shape=jax.ShapeDtypeStruct((1, 1), jnp.float32),
)


def kernel(alpha, beta, edge_index0, edge_index1, edge_index2, score0, score1):
    del alpha, beta
    edges = [
        e.astype(jnp.int32)
        for e in (edge_index0, edge_index1, edge_index2, score0, score1)
    ]
    counts = _make_scan()(*edges)
    return _finish(counts)[0, 0]


# speedup vs baseline: 1.6916x; 1.0070x over previous
import functools

import jax
import jax.numpy as jnp
from jax import lax
from jax.experimental import pallas as pl
from jax.experimental.pallas import tpu as pltpu
from jax.experimental.pallas import tpu_sc as plsc

B = 4096
E = 131072

NC = 2
NS = 16
NW = NC * NS

NSTRIP = 16
ROWS = B // NSTRIP
WPP = ROWS * B // 32
BMW = 3 * WPP

ECH = 4096
NCH = E // ECH
LANES = 16
RCH = 2048


def _any(m):
    return plsc.all_reduce_population_count(m)[0] > 0


def _rmw_or(bm, w, bitv, m):
    def round_(lost):
        old = plsc.load_gather(bm, [w], mask=lost)
        plsc.store_scatter(bm, [w], old | bitv, mask=lost)
        chk = plsc.load_gather(bm, [w], mask=lost)
        return lost & ((chk & bitv) != bitv)

    lost = round_(m)
    lax.while_loop(_any, round_, lost)


def _halfpop(v):
    v = v - ((v >> 1) & 0x55555555)
    v = (v & 0x33333333) + ((v >> 2) & 0x33333333)
    return (v + (v >> 4)) & 0x0F0F0F0F


def _scan_body(
    e0, e1, e2, e3, e4, out, ebuf_a, ebuf_b, accbuf, bm, st0, st1, st2, spb, sem_a, sem_b
):
    cid = lax.axis_index("c")
    sub = lax.axis_index("s")
    wid = sub * NC + cid
    pair = sub >> 1
    strip = cid * 8 + pair
    role = sub & 1

    def zero(i, _):
        bm[pl.ds(i * LANES, LANES)] = jnp.zeros((LANES,), jnp.int32)
        return 0

    lax.fori_loop(0, BMW // LANES, zero, 0)

    one = jnp.full((LANES,), 1, jnp.int32)

    def process(ebuf, pbase):
        def it(t, lostacc):
            r = ebuf[0, pl.ds(t * LANES, LANES)]
            c = ebuf[1, pl.ds(t * LANES, LANES)]
            for x, y in ((r, c), (c, r)):
                m = (x >> 8) == strip
                w = pbase | ((x & 255) << 7) | (y >> 5)
                bitv = jnp.left_shift(one, y & 31)
                old = plsc.load_gather(bm, [w], mask=m)
                plsc.store_scatter(bm, [w], old | bitv, mask=m)
                chk = plsc.load_gather(bm, [w], mask=m)
                lostacc = lostacc | (m & ((chk & bitv) != bitv))
            return lostacc

        lost = lax.fori_loop(
            0, ECH // LANES, it, jnp.zeros((LANES,), jnp.bool_), unroll=4
        )

        @pl.when(_any(lost))
        def _():
            def it2(t, _):
                r = ebuf[0, pl.ds(t * LANES, LANES)]
                c = ebuf[1, pl.ds(t * LANES, LANES)]
                for x, y in ((r, c), (c, r)):
                    m = (x >> 8) == strip
                    w = pbase | ((x & 255) << 7) | (y >> 5)
                    bitv = jnp.left_shift(one, y & 31)
                    _rmw_or(bm, w, bitv, m)
                return 0

            lax.fori_loop(0, ECH // LANES, it2, 0)

    def scan_list(e, pbase, lo, hi):
        pltpu.async_copy(e.at[:, pl.ds(lo * ECH, ECH)], ebuf_a, sem_a)

        def pairstep(pi, _):
            i0 = lo + pi * 2
            pltpu.make_async_copy(
                e.at[:, pl.ds(i0 * ECH, ECH)], ebuf_a, sem_a
            ).wait()
            pltpu.async_copy(e.at[:, pl.ds((i0 + 1) * ECH, ECH)], ebuf_b, sem_b)
            process(ebuf_a, pbase)
            pltpu.make_async_copy(
                e.at[:, pl.ds((i0 + 1) * ECH, ECH)], ebuf_b, sem_b
            ).wait()

            @pl.when(i0 + 2 < hi)
            def _():
                pltpu.async_copy(
                    e.at[:, pl.ds((i0 + 2) * ECH, ECH)], ebuf_a, sem_a
                )

            process(ebuf_b, pbase)
            return 0

        lax.fori_loop(0, (hi - lo) // 2, pairstep, 0)

    @pl.when(role == 0)
    def _():
        scan_list(e0, 0, 0, NCH)
        scan_list(e1, WPP, 0, NCH)
        scan_list(e2, 2 * WPP, 0, NCH // 2)

    @pl.when(role == 1)
    def _():
        scan_list(e3, 0, 0, NCH)
        scan_list(e4, WPP, 0, NCH)
        scan_list(e2, 2 * WPP, NCH // 2, NCH)

    @pl.when(role == 1)
    def _():
        pltpu.sync_copy(bm, spb.at[strip])
        accbuf[...] = jnp.zeros((LANES,), jnp.int32)
        pltpu.sync_copy(accbuf, out.at[wid])

    plsc.subcore_barrier()

    @pl.when(role == 0)
    def _():
        def chunk(ch, acc):
            off = ch * RCH
            pltpu.sync_copy(spb.at[strip, pl.ds(off, RCH)], st0)
            pltpu.sync_copy(spb.at[strip, pl.ds(WPP + off, RCH)], st1)
            pltpu.sync_copy(spb.at[strip, pl.ds(2 * WPP + off, RCH)], st2)

            def red(g, acc2):
                o = g * LANES
                a0 = bm[pl.ds(off + o, LANES)]
                a1 = bm[pl.ds(WPP + off + o, LANES)]
                a2 = bm[pl.ds(2 * WPP + off + o, LANES)] | st2[pl.ds(o, LANES)]
                s0 = st0[pl.ds(o, LANES)]
                s1 = st1[pl.ds(o, LANES)]
                d0 = a0 & ~a1 & ~a2
                d1 = a1 & ~a0 & ~a2
                tot = _halfpop(d0 ^ s0) + _halfpop(d1 ^ s1)
                return acc2 + ((tot * 0x01010101) >> 24)

            return lax.fori_loop(0, RCH // LANES, red, acc)

        acc = lax.fori_loop(0, WPP // RCH, chunk, jnp.zeros((LANES,), jnp.int32))
        accbuf[...] = acc
        pltpu.sync_copy(accbuf, out.at[wid])


@functools.cache
def _make_scan():
    return pl.kernel(
        _scan_body,
        out_type=jax.ShapeDtypeStruct((NW, LANES), jnp.int32),
        mesh=plsc.VectorSubcoreMesh(
            core_axis_name="c",
            subcore_axis_name="s",
            num_cores=NC,
            num_subcores=NS,
        ),
        compiler_params=pltpu.CompilerParams(needs_layout_passes=False),
        scratch_types=[
            pltpu.VMEM((2, ECH), jnp.int32),
            pltpu.VMEM((2, ECH), jnp.int32),
            pltpu.VMEM((LANES,), jnp.int32),
            pltpu.VMEM((BMW,), jnp.int32),
            pltpu.VMEM((RCH,), jnp.int32),
            pltpu.VMEM((RCH,), jnp.int32),
            pltpu.VMEM((RCH,), jnp.int32),
            pltpu.HBM((NSTRIP, BMW), jnp.int32),
            pltpu.SemaphoreType.DMA,
            pltpu.SemaphoreType.DMA,
        ],
    )


def _finish_body(x_ref, o_ref):
    o_ref[0, 0] = jnp.sum(x_ref[...]).astype(jnp.float32) * (1.0 / (2 * B * B))


_finish = pl.pallas_call(
    _finish_body,
    out_specs=pl.BlockSpec(memory_space=pltpu.SMEM),
    out_shape=jax.ShapeDtypeStruct((1, 1), jnp.float32),
)


def kernel(alpha, beta, edge_index0, edge_index1, edge_index2, score0, score1):
    del alpha, beta
    edges = [
        e.astype(jnp.int32)
        for e in (edge_index0, edge_index1, edge_index2, score0, score1)
    ]
    counts = _make_scan()(*edges)
    return _finish(counts)[0, 0]
